# Initial kernel scaffold; baseline (speedup 1.0000x reference)
import jax
import jax.numpy as jnp
from jax.experimental import pallas as pl


def kernel(tgt, memory, query_pos, key_pos, params):
    raise NotImplementedError("write your pallas kernel here")


# speedup vs baseline: 3.9074x; 3.9074x over previous
import math

import jax
import jax.numpy as jnp
from jax import lax
from jax.experimental import pallas as pl

BS, NP, MNP, D, NH, DFF, NC, TOPK = 2, 256, 4096, 768, 12, 2048, 751, 32
DH = D // NH
F32 = jnp.float32
NEG = -1000000000.0


def _dot(a, b):
    return jnp.dot(a, b, preferred_element_type=F32)


def _dot_t(a, b):
    return lax.dot_general(a, b, (((1,), (1,)), ((), ())),
                           preferred_element_type=F32)


def _ln(x, g, b, eps=1e-5):
    mu = jnp.mean(x, axis=-1, keepdims=True)
    var = jnp.mean((x - mu) ** 2, axis=-1, keepdims=True)
    return (x - mu) / jnp.sqrt(var + eps) * g + b


def _w2(p):
    return p.reshape(1, -1)


def _ln0_body(x_ref, g_ref, b_ref, o_ref):
    o_ref[0] = _ln(x_ref[0], g_ref[...], b_ref[...])


def _mn_body(m_ref, mn_ref):
    m = m_ref[0]
    n = jnp.sqrt(jnp.sum(m * m, axis=1, keepdims=True))
    mn_ref[0] = m / n


def _kv_body(m_ref, kp_ref, w2_ref, b2_ref, w3_ref, b3_ref, k_ref, v_ref):
    m = m_ref[0]
    k_ref[0] = _dot(m + kp_ref[0], w2_ref[...]) + b2_ref[...]
    v_ref[0] = _dot(m, w3_ref[...]) + b3_ref[...]


def _sim_body(x_ref, mn_ref, sim_ref):
    x = x_ref[0]
    xn = x / jnp.sqrt(jnp.sum(x * x, axis=1, keepdims=True))
    sim_ref[0] = _dot_t(xn, mn_ref[0])


def _topk_body(sim_ref, mask_ref):
    sim = sim_ref[0]
    iota = lax.broadcasted_iota(jnp.int32, sim.shape, 1)

    def body(_, sim_c):
        m = jnp.max(sim_c, axis=1, keepdims=True)
        idx = jnp.min(jnp.where(sim_c == m, iota, MNP), axis=1, keepdims=True)
        return jnp.where(iota == idx, jnp.float32(-3e38), sim_c)

    sim_c = lax.fori_loop(0, TOPK, body, sim)
    mask_ref[0] = jnp.where(sim_c <= -1e38, 1.0, 0.0).astype(F32)


def _q_body(x_ref, qp_ref, w_ref, b_ref, q_ref):
    q_ref[0] = _dot(x_ref[0] + qp_ref[0], w_ref[...]) + b_ref[...]


def _xattn_body(q_ref, k_ref, v_ref, mask_ref, o_ref):
    s = _dot_t(q_ref[0, 0], k_ref[0, 0]) * (1.0 / math.sqrt(DH))
    mask = mask_ref[0]
    s = s * mask + NEG * (1.0 - mask)
    m = jnp.max(s, axis=1, keepdims=True)
    e = jnp.exp(s - m)
    w = e / jnp.sum(e, axis=1, keepdims=True)
    o_ref[0, 0] = _dot(w, v_ref[0, 0])


def _post1_body(x_ref, att_ref, qp_ref, g1_ref, b1_ref, wq_ref, bq_ref,
                wk_ref, bk_ref, wv_ref, bv_ref,
                t1_ref, qs_ref, ks_ref, vs_ref):
    t1 = _ln(x_ref[0] + att_ref[0], g1_ref[...], b1_ref[...])
    qk = t1 + qp_ref[0]
    t1_ref[0] = t1
    qs_ref[0] = _dot(qk, wq_ref[...]) + bq_ref[...]
    ks_ref[0] = _dot(qk, wk_ref[...]) + bk_ref[...]
    vs_ref[0] = _dot(t1, wv_ref[...]) + bv_ref[...]


def _sa_body(q_ref, k_ref, v_ref, o_ref):
    s = _dot_t(q_ref[0, 0], k_ref[0, 0]) * (1.0 / math.sqrt(DH))
    m = jnp.max(s, axis=1, keepdims=True)
    e = jnp.exp(s - m)
    w = e / jnp.sum(e, axis=1, keepdims=True)
    o_ref[0, 0] = _dot(w, v_ref[0, 0])


def _post2_body(t1_ref, sa_ref, wo_ref, bo_ref, g2_ref, b2_ref,
                w1_ref, bf1_ref, w2_ref, bf2_ref, g3_ref, b3_ref, o_ref):
    sao = _dot(sa_ref[0], wo_ref[...]) + bo_ref[...]
    t2 = _ln(t1_ref[0] + sao, g2_ref[...], b2_ref[...])
    h = _dot(t2, w1_ref[...]) + bf1_ref[...]
    h = 0.5 * h * (1.0 + lax.erf(h * (1.0 / math.sqrt(2.0))))
    ff = _dot(h, w2_ref[...]) + bf2_ref[...]
    o_ref[0] = _ln(t2 + ff, g3_ref[...], b3_ref[...])


def _head_body(x_ref, g_ref, b_ref, rm_ref, rv_ref, w_ref, c_ref):
    x = (x_ref[0] - rm_ref[...]) / jnp.sqrt(rv_ref[...] + 1e-5) \
        * g_ref[...] + b_ref[...]
    c_ref[0] = _dot(x, w_ref[...])


def _full(shape):
    nd = len(shape)
    return pl.BlockSpec(shape, lambda *_: (0,) * nd)


def _bmap(shape):
    nd = len(shape)
    return pl.BlockSpec(shape, lambda b, *_: (b,) + (0,) * (nd - 1))


def _rows_call(body, ins, n_out=1, out_shape=None):
    in_specs = [_bmap((1,) + a.shape[1:]) if a.shape[0] == BS else _full(a.shape)
                for a in ins]
    if out_shape is None:
        out_shape = (BS, NP, D)
    shapes = [out_shape] * n_out if isinstance(out_shape, tuple) else out_shape
    out_specs = [_bmap((1,) + s[1:]) for s in shapes]
    res = pl.pallas_call(
        body,
        grid=(BS,),
        in_specs=in_specs,
        out_specs=out_specs if n_out > 1 else out_specs[0],
        out_shape=[jax.ShapeDtypeStruct(s, F32) for s in shapes]
        if n_out > 1 else jax.ShapeDtypeStruct(shapes[0], F32),
    )(*ins)
    return res


def _heads(x):
    return x.reshape(BS, -1, NH, DH).transpose(0, 2, 1, 3)


def _unheads(x):
    return x.transpose(0, 2, 1, 3).reshape(BS, -1, D)


def _attn_call(q4, k4, v4, mask):
    return pl.pallas_call(
        _xattn_body,
        grid=(BS, NH),
        in_specs=[
            pl.BlockSpec((1, 1, NP, DH), lambda b, h: (b, h, 0, 0)),
            pl.BlockSpec((1, 1, MNP, DH), lambda b, h: (b, h, 0, 0)),
            pl.BlockSpec((1, 1, MNP, DH), lambda b, h: (b, h, 0, 0)),
            pl.BlockSpec((1, NP, MNP), lambda b, h: (b, 0, 0)),
        ],
        out_specs=pl.BlockSpec((1, 1, NP, DH), lambda b, h: (b, h, 0, 0)),
        out_shape=jax.ShapeDtypeStruct((BS, NH, NP, DH), F32),
    )(q4, k4, v4, mask)


def _sa_call(qs4, ks4, vs4):
    return pl.pallas_call(
        _sa_body,
        grid=(BS, NH),
        in_specs=[pl.BlockSpec((1, 1, NP, DH), lambda b, h: (b, h, 0, 0))] * 3,
        out_specs=pl.BlockSpec((1, 1, NP, DH), lambda b, h: (b, h, 0, 0)),
        out_shape=jax.ShapeDtypeStruct((BS, NH, NP, DH), F32),
    )(qs4, ks4, vs4)


def kernel(tgt, memory, query_pos, key_pos, params):
    p = params

    x = _rows_call(_ln0_body,
                   [tgt, _w2(p["norm0"]["g"]), _w2(p["norm0"]["b"])])

    mrows = 512
    mn = pl.pallas_call(
        _mn_body,
        grid=(BS, MNP // mrows),
        in_specs=[pl.BlockSpec((1, mrows, D), lambda b, i: (b, i, 0))],
        out_specs=pl.BlockSpec((1, mrows, D), lambda b, i: (b, i, 0)),
        out_shape=jax.ShapeDtypeStruct((BS, MNP, D), F32),
    )(memory)

    for lp in p["layers"]:
        k, v = pl.pallas_call(
            _kv_body,
            grid=(BS, MNP // mrows),
            in_specs=[
                pl.BlockSpec((1, mrows, D), lambda b, i: (b, i, 0)),
                pl.BlockSpec((1, mrows, D), lambda b, i: (b, i, 0)),
                _full((D, D)), _full((1, D)), _full((D, D)), _full((1, D)),
            ],
            out_specs=[pl.BlockSpec((1, mrows, D), lambda b, i: (b, i, 0))] * 2,
            out_shape=[jax.ShapeDtypeStruct((BS, MNP, D), F32)] * 2,
        )(memory, key_pos, lp["l2"]["W"], _w2(lp["l2"]["b"]),
          lp["l3"]["W"], _w2(lp["l3"]["b"]))

        mcols = 1024
        sim = pl.pallas_call(
            _sim_body,
            grid=(BS, MNP // mcols),
            in_specs=[pl.BlockSpec((1, NP, D), lambda b, i: (b, 0, 0)),
                      pl.BlockSpec((1, mcols, D), lambda b, i: (b, i, 0))],
            out_specs=pl.BlockSpec((1, NP, mcols), lambda b, i: (b, 0, i)),
            out_shape=jax.ShapeDtypeStruct((BS, NP, MNP), F32),
        )(x, mn)
        mask = _rows_call(_topk_body, [sim], out_shape=(BS, NP, MNP))

        q = _rows_call(_q_body,
                       [x, query_pos, lp["l1"]["W"], _w2(lp["l1"]["b"])])
        att = _unheads(_attn_call(_heads(q), _heads(k), _heads(v), mask))

        sa = lp["sa"]
        t1, qs, ks, vs = _rows_call(
            _post1_body,
            [x, att, query_pos, _w2(lp["norm1"]["g"]), _w2(lp["norm1"]["b"]),
             sa["Wq"], _w2(sa["bq"]), sa["Wk"], _w2(sa["bk"]),
             sa["Wv"], _w2(sa["bv"])],
            n_out=4)
        sao = _unheads(_sa_call(_heads(qs), _heads(ks), _heads(vs)))

        x = _rows_call(
            _post2_body,
            [t1, sao, sa["Wo"], _w2(sa["bo"]),
             _w2(lp["norm2"]["g"]), _w2(lp["norm2"]["b"]),
             lp["lin1"]["W"], _w2(lp["lin1"]["b"]),
             lp["lin2"]["W"], _w2(lp["lin2"]["b"]),
             _w2(lp["norm3"]["g"]), _w2(lp["norm3"]["b"])])

    ncp = 768
    cls_w = jnp.pad(p["cls_W"], ((0, 0), (0, ncp - NC)))
    bn = p["bn"]
    cls = _rows_call(
        _head_body,
        [x, _w2(bn["g"]), _w2(bn["b"]), _w2(bn["rm"]), _w2(bn["rv"]), cls_w],
        out_shape=(BS, NP, ncp))
    return (x, cls[..., :NC])


# speedup vs baseline: 5.4158x; 1.3861x over previous
import math

import jax
import jax.numpy as jnp
from jax import lax
from jax.experimental import pallas as pl

BS, NP, MNP, D, NH, DFF, NC, TOPK = 2, 256, 4096, 768, 12, 2048, 751, 32
DH = D // NH
F32 = jnp.float32
NEG = -1000000000.0


def _dot(a, b):
    return jnp.dot(a, b, preferred_element_type=F32)


def _dot_t(a, b):
    return lax.dot_general(a, b, (((1,), (1,)), ((), ())),
                           preferred_element_type=F32)


def _ln(x, g, b, eps=1e-5):
    mu = jnp.mean(x, axis=-1, keepdims=True)
    var = jnp.mean((x - mu) ** 2, axis=-1, keepdims=True)
    return (x - mu) / jnp.sqrt(var + eps) * g + b


def _w2(p):
    return p.reshape(1, -1)


def _ln0_body(x_ref, g_ref, b_ref, o_ref):
    o_ref[0] = _ln(x_ref[0], g_ref[...], b_ref[...])


def _mn_body(m_ref, mn_ref):
    m = m_ref[0]
    n = jnp.sqrt(jnp.sum(m * m, axis=1, keepdims=True))
    mn_ref[0] = m / n


def _kv_body(m_ref, kp_ref, w2_ref, b2_ref, w3_ref, b3_ref, k_ref, v_ref):
    m = m_ref[0]
    k_ref[0] = _dot(m + kp_ref[0], w2_ref[...]) + b2_ref[...]
    v_ref[0] = _dot(m, w3_ref[...]) + b3_ref[...]


def _sim_body(x_ref, mn_ref, sim_ref):
    x = x_ref[0]
    xn = x / jnp.sqrt(jnp.sum(x * x, axis=1, keepdims=True))
    sim_ref[0] = _dot_t(xn, mn_ref[0])


def _topk_body(sim_ref, mask_ref):
    sim = sim_ref[0]
    iota = lax.broadcasted_iota(jnp.int32, sim.shape, 1)

    def body(_, sim_c):
        m = jnp.max(sim_c, axis=1, keepdims=True)
        idx = jnp.min(jnp.where(sim_c == m, iota, MNP), axis=1, keepdims=True)
        return jnp.where(iota == idx, jnp.float32(-3e38), sim_c)

    sim_c = lax.fori_loop(0, TOPK, body, sim)
    mask_ref[0] = jnp.where(sim_c <= -1e38, 1.0, 0.0).astype(F32)


def _q_body(x_ref, qp_ref, w_ref, b_ref, q_ref):
    q_ref[0] = _dot(x_ref[0] + qp_ref[0], w_ref[...]) + b_ref[...]


def _softmax(s):
    m = jnp.max(s, axis=1, keepdims=True)
    e = jnp.exp(s - m)
    return e / jnp.sum(e, axis=1, keepdims=True)


def _xattn_body(q_ref, k_ref, v_ref, mask_ref, o_ref):
    mask = mask_ref[0]
    neg = NEG * (1.0 - mask)
    outs = []
    for i in range(2):
        sl = slice(i * DH, (i + 1) * DH)
        s = _dot_t(q_ref[0][:, sl], k_ref[0][:, sl]) * (1.0 / math.sqrt(DH))
        w = _softmax(s * mask + neg)
        outs.append(_dot(w, v_ref[0][:, sl]))
    o_ref[0] = jnp.concatenate(outs, axis=1)


def _post1_body(x_ref, att_ref, qp_ref, g1_ref, b1_ref, wq_ref, bq_ref,
                wk_ref, bk_ref, wv_ref, bv_ref,
                t1_ref, qs_ref, ks_ref, vs_ref):
    t1 = _ln(x_ref[0] + att_ref[0], g1_ref[...], b1_ref[...])
    qk = t1 + qp_ref[0]
    t1_ref[0] = t1
    qs_ref[0] = _dot(qk, wq_ref[...]) + bq_ref[...]
    ks_ref[0] = _dot(qk, wk_ref[...]) + bk_ref[...]
    vs_ref[0] = _dot(t1, wv_ref[...]) + bv_ref[...]


def _sa_body(q_ref, k_ref, v_ref, o_ref):
    outs = []
    for i in range(2):
        sl = slice(i * DH, (i + 1) * DH)
        s = _dot_t(q_ref[0][:, sl], k_ref[0][:, sl]) * (1.0 / math.sqrt(DH))
        outs.append(_dot(_softmax(s), v_ref[0][:, sl]))
    o_ref[0] = jnp.concatenate(outs, axis=1)


def _post2_body(t1_ref, sa_ref, wo_ref, bo_ref, g2_ref, b2_ref,
                w1_ref, bf1_ref, w2_ref, bf2_ref, g3_ref, b3_ref, o_ref):
    sao = _dot(sa_ref[0], wo_ref[...]) + bo_ref[...]
    t2 = _ln(t1_ref[0] + sao, g2_ref[...], b2_ref[...])
    h = _dot(t2, w1_ref[...]) + bf1_ref[...]
    h = 0.5 * h * (1.0 + lax.erf(h * (1.0 / math.sqrt(2.0))))
    ff = _dot(h, w2_ref[...]) + bf2_ref[...]
    o_ref[0] = _ln(t2 + ff, g3_ref[...], b3_ref[...])


def _head_body(x_ref, g_ref, b_ref, rm_ref, rv_ref, w_ref, c_ref):
    x = (x_ref[0] - rm_ref[...]) / jnp.sqrt(rv_ref[...] + 1e-5) \
        * g_ref[...] + b_ref[...]
    c_ref[0] = _dot(x, w_ref[...])


def _full(shape):
    nd = len(shape)
    return pl.BlockSpec(shape, lambda *_: (0,) * nd)


def _bmap(shape):
    nd = len(shape)
    return pl.BlockSpec(shape, lambda b, *_: (b,) + (0,) * (nd - 1))


def _rows_call(body, ins, n_out=1, out_shape=None):
    in_specs = [_bmap((1,) + a.shape[1:]) if a.shape[0] == BS else _full(a.shape)
                for a in ins]
    if out_shape is None:
        out_shape = (BS, NP, D)
    shapes = [out_shape] * n_out if isinstance(out_shape, tuple) else out_shape
    out_specs = [_bmap((1,) + s[1:]) for s in shapes]
    res = pl.pallas_call(
        body,
        grid=(BS,),
        in_specs=in_specs,
        out_specs=out_specs if n_out > 1 else out_specs[0],
        out_shape=[jax.ShapeDtypeStruct(s, F32) for s in shapes]
        if n_out > 1 else jax.ShapeDtypeStruct(shapes[0], F32),
    )(*ins)
    return res


def _attn_call(q, k, v, mask):
    hd = 2 * DH
    return pl.pallas_call(
        _xattn_body,
        grid=(BS, NH // 2),
        in_specs=[
            pl.BlockSpec((1, NP, hd), lambda b, h: (b, 0, h)),
            pl.BlockSpec((1, MNP, hd), lambda b, h: (b, 0, h)),
            pl.BlockSpec((1, MNP, hd), lambda b, h: (b, 0, h)),
            pl.BlockSpec((1, NP, MNP), lambda b, h: (b, 0, 0)),
        ],
        out_specs=pl.BlockSpec((1, NP, hd), lambda b, h: (b, 0, h)),
        out_shape=jax.ShapeDtypeStruct((BS, NP, D), F32),
    )(q, k, v, mask)


def _sa_call(qs, ks, vs):
    hd = 2 * DH
    return pl.pallas_call(
        _sa_body,
        grid=(BS, NH // 2),
        in_specs=[pl.BlockSpec((1, NP, hd), lambda b, h: (b, 0, h))] * 3,
        out_specs=pl.BlockSpec((1, NP, hd), lambda b, h: (b, 0, h)),
        out_shape=jax.ShapeDtypeStruct((BS, NP, D), F32),
    )(qs, ks, vs)


def kernel(tgt, memory, query_pos, key_pos, params):
    p = params

    x = _rows_call(_ln0_body,
                   [tgt, _w2(p["norm0"]["g"]), _w2(p["norm0"]["b"])])

    mrows = 512
    mn = pl.pallas_call(
        _mn_body,
        grid=(BS, MNP // mrows),
        in_specs=[pl.BlockSpec((1, mrows, D), lambda b, i: (b, i, 0))],
        out_specs=pl.BlockSpec((1, mrows, D), lambda b, i: (b, i, 0)),
        out_shape=jax.ShapeDtypeStruct((BS, MNP, D), F32),
    )(memory)

    for lp in p["layers"]:
        k, v = pl.pallas_call(
            _kv_body,
            grid=(BS, MNP // mrows),
            in_specs=[
                pl.BlockSpec((1, mrows, D), lambda b, i: (b, i, 0)),
                pl.BlockSpec((1, mrows, D), lambda b, i: (b, i, 0)),
                _full((D, D)), _full((1, D)), _full((D, D)), _full((1, D)),
            ],
            out_specs=[pl.BlockSpec((1, mrows, D), lambda b, i: (b, i, 0))] * 2,
            out_shape=[jax.ShapeDtypeStruct((BS, MNP, D), F32)] * 2,
        )(memory, key_pos, lp["l2"]["W"], _w2(lp["l2"]["b"]),
          lp["l3"]["W"], _w2(lp["l3"]["b"]))

        mcols = 1024
        sim = pl.pallas_call(
            _sim_body,
            grid=(BS, MNP // mcols),
            in_specs=[pl.BlockSpec((1, NP, D), lambda b, i: (b, 0, 0)),
                      pl.BlockSpec((1, mcols, D), lambda b, i: (b, i, 0))],
            out_specs=pl.BlockSpec((1, NP, mcols), lambda b, i: (b, 0, i)),
            out_shape=jax.ShapeDtypeStruct((BS, NP, MNP), F32),
        )(x, mn)
        mask = _rows_call(_topk_body, [sim], out_shape=(BS, NP, MNP))

        q = _rows_call(_q_body,
                       [x, query_pos, lp["l1"]["W"], _w2(lp["l1"]["b"])])
        att = _attn_call(q, k, v, mask)

        sa = lp["sa"]
        t1, qs, ks, vs = _rows_call(
            _post1_body,
            [x, att, query_pos, _w2(lp["norm1"]["g"]), _w2(lp["norm1"]["b"]),
             sa["Wq"], _w2(sa["bq"]), sa["Wk"], _w2(sa["bk"]),
             sa["Wv"], _w2(sa["bv"])],
            n_out=4)
        sao = _sa_call(qs, ks, vs)

        x = _rows_call(
            _post2_body,
            [t1, sao, sa["Wo"], _w2(sa["bo"]),
             _w2(lp["norm2"]["g"]), _w2(lp["norm2"]["b"]),
             lp["lin1"]["W"], _w2(lp["lin1"]["b"]),
             lp["lin2"]["W"], _w2(lp["lin2"]["b"]),
             _w2(lp["norm3"]["g"]), _w2(lp["norm3"]["b"])])

    ncp = 768
    cls_w = jnp.pad(p["cls_W"], ((0, 0), (0, ncp - NC)))
    bn = p["bn"]
    cls = _rows_call(
        _head_body,
        [x, _w2(bn["g"]), _w2(bn["b"]), _w2(bn["rm"]), _w2(bn["rv"]), cls_w],
        out_shape=(BS, NP, ncp))
    return (x, cls[..., :NC])


# speedup vs baseline: 7.2223x; 1.3335x over previous
import math

import jax
import jax.numpy as jnp
from jax import lax
from jax.experimental import pallas as pl

BS, NP, MNP, D, NH, DFF, NC, TOPK = 2, 256, 4096, 768, 12, 2048, 751, 32
DH = D // NH
F32 = jnp.float32
NEG = -1000000000.0


def _dot(a, b):
    return jnp.dot(a, b, preferred_element_type=F32)


def _dot_t(a, b):
    return lax.dot_general(a, b, (((1,), (1,)), ((), ())),
                           preferred_element_type=F32)


def _ln(x, g, b, eps=1e-5):
    mu = jnp.mean(x, axis=-1, keepdims=True)
    var = jnp.mean((x - mu) ** 2, axis=-1, keepdims=True)
    return (x - mu) / jnp.sqrt(var + eps) * g + b


def _w2(p):
    return p.reshape(1, -1)


def _ln0_body(x_ref, g_ref, b_ref, o_ref):
    o_ref[0] = _ln(x_ref[0], g_ref[...], b_ref[...])


def _mn_body(m_ref, mn_ref):
    m = m_ref[0]
    n = jnp.sqrt(jnp.sum(m * m, axis=1, keepdims=True))
    mn_ref[0] = m / n


def _kv_body(m_ref, kp_ref, w2_ref, b2_ref, w3_ref, b3_ref, k_ref, v_ref):
    m = m_ref[0]
    k_ref[0] = _dot(m + kp_ref[0], w2_ref[...]) + b2_ref[...]
    v_ref[0] = _dot(m, w3_ref[...]) + b3_ref[...]


def _sim_body(x_ref, mn_ref, sim_ref):
    x = x_ref[0]
    xn = x / jnp.sqrt(jnp.sum(x * x, axis=1, keepdims=True))
    sim_ref[0] = _dot_t(xn, mn_ref[0])


def _topk_body(sim_ref, mask_ref):
    sim = sim_ref[0]
    b = lax.bitcast_convert_type(sim, jnp.int32)
    key = b ^ (lax.shift_right_arithmetic(b, 31) & jnp.int32(0x7FFFFFFF))
    sign = jnp.int32(-2147483648)

    def body(_, carry):
        t_u, bit = carry
        cand = t_u | bit
        thr = cand ^ sign
        cnt = jnp.sum((key >= thr).astype(jnp.int32), axis=1, keepdims=True)
        return jnp.where(cnt >= TOPK, cand, t_u), lax.shift_right_logical(bit, 1)

    t_u, _ = lax.fori_loop(
        0, 32, body, (jnp.zeros((NP, 1), jnp.int32), jnp.full((), sign)))
    thr = t_u ^ sign
    gt = key > thr
    n_gt = jnp.sum(gt.astype(jnp.int32), axis=1, keepdims=True)
    eq = key == thr
    nb = MNP // 128
    eq2 = eq.astype(F32).reshape(NP * nb, 128)
    li = lax.broadcasted_iota(jnp.int32, (128, 128), 0)
    lj = lax.broadcasted_iota(jnp.int32, (128, 128), 1)
    inb = _dot(eq2, (li <= lj).astype(F32)).reshape(NP, nb, 128)
    btot = jnp.sum(eq.astype(F32).reshape(NP, nb, 128), axis=2)
    ci = lax.broadcasted_iota(jnp.int32, (nb, nb), 0)
    cj = lax.broadcasted_iota(jnp.int32, (nb, nb), 1)
    offs = _dot(btot, (ci < cj).astype(F32))
    incl = (inb + offs[:, :, None]).reshape(NP, MNP)
    need = (TOPK - n_gt).astype(F32)
    sel = eq & (incl <= need)
    mask_ref[0] = jnp.where(gt | sel, 1.0, 0.0).astype(F32)


def _q_body(x_ref, qp_ref, w_ref, b_ref, q_ref):
    q_ref[0] = _dot(x_ref[0] + qp_ref[0], w_ref[...]) + b_ref[...]


def _softmax(s):
    m = jnp.max(s, axis=1, keepdims=True)
    e = jnp.exp(s - m)
    return e / jnp.sum(e, axis=1, keepdims=True)


def _xattn_body(q_ref, k_ref, v_ref, mask_ref, o_ref):
    mask = mask_ref[0]
    neg = NEG * (1.0 - mask)
    outs = []
    for i in range(2):
        sl = slice(i * DH, (i + 1) * DH)
        s = _dot_t(q_ref[0][:, sl], k_ref[0][:, sl]) * (1.0 / math.sqrt(DH))
        w = _softmax(s * mask + neg)
        outs.append(_dot(w, v_ref[0][:, sl]))
    o_ref[0] = jnp.concatenate(outs, axis=1)


def _post1_body(x_ref, att_ref, qp_ref, g1_ref, b1_ref, wq_ref, bq_ref,
                wk_ref, bk_ref, wv_ref, bv_ref,
                t1_ref, qs_ref, ks_ref, vs_ref):
    t1 = _ln(x_ref[0] + att_ref[0], g1_ref[...], b1_ref[...])
    qk = t1 + qp_ref[0]
    t1_ref[0] = t1
    qs_ref[0] = _dot(qk, wq_ref[...]) + bq_ref[...]
    ks_ref[0] = _dot(qk, wk_ref[...]) + bk_ref[...]
    vs_ref[0] = _dot(t1, wv_ref[...]) + bv_ref[...]


def _sa_body(q_ref, k_ref, v_ref, o_ref):
    outs = []
    for i in range(2):
        sl = slice(i * DH, (i + 1) * DH)
        s = _dot_t(q_ref[0][:, sl], k_ref[0][:, sl]) * (1.0 / math.sqrt(DH))
        outs.append(_dot(_softmax(s), v_ref[0][:, sl]))
    o_ref[0] = jnp.concatenate(outs, axis=1)


def _post2_body(t1_ref, sa_ref, wo_ref, bo_ref, g2_ref, b2_ref,
                w1_ref, bf1_ref, w2_ref, bf2_ref, g3_ref, b3_ref, o_ref):
    sao = _dot(sa_ref[0], wo_ref[...]) + bo_ref[...]
    t2 = _ln(t1_ref[0] + sao, g2_ref[...], b2_ref[...])
    h = _dot(t2, w1_ref[...]) + bf1_ref[...]
    h = 0.5 * h * (1.0 + lax.erf(h * (1.0 / math.sqrt(2.0))))
    ff = _dot(h, w2_ref[...]) + bf2_ref[...]
    o_ref[0] = _ln(t2 + ff, g3_ref[...], b3_ref[...])


def _head_body(x_ref, g_ref, b_ref, rm_ref, rv_ref, w_ref, c_ref):
    x = (x_ref[0] - rm_ref[...]) / jnp.sqrt(rv_ref[...] + 1e-5) \
        * g_ref[...] + b_ref[...]
    c_ref[0] = _dot(x, w_ref[...])


def _full(shape):
    nd = len(shape)
    return pl.BlockSpec(shape, lambda *_: (0,) * nd)


def _bmap(shape):
    nd = len(shape)
    return pl.BlockSpec(shape, lambda b, *_: (b,) + (0,) * (nd - 1))


def _rows_call(body, ins, n_out=1, out_shape=None):
    in_specs = [_bmap((1,) + a.shape[1:]) if a.shape[0] == BS else _full(a.shape)
                for a in ins]
    if out_shape is None:
        out_shape = (BS, NP, D)
    shapes = [out_shape] * n_out if isinstance(out_shape, tuple) else out_shape
    out_specs = [_bmap((1,) + s[1:]) for s in shapes]
    res = pl.pallas_call(
        body,
        grid=(BS,),
        in_specs=in_specs,
        out_specs=out_specs if n_out > 1 else out_specs[0],
        out_shape=[jax.ShapeDtypeStruct(s, F32) for s in shapes]
        if n_out > 1 else jax.ShapeDtypeStruct(shapes[0], F32),
    )(*ins)
    return res


def _attn_call(q, k, v, mask):
    hd = 2 * DH
    return pl.pallas_call(
        _xattn_body,
        grid=(BS, NH // 2),
        in_specs=[
            pl.BlockSpec((1, NP, hd), lambda b, h: (b, 0, h)),
            pl.BlockSpec((1, MNP, hd), lambda b, h: (b, 0, h)),
            pl.BlockSpec((1, MNP, hd), lambda b, h: (b, 0, h)),
            pl.BlockSpec((1, NP, MNP), lambda b, h: (b, 0, 0)),
        ],
        out_specs=pl.BlockSpec((1, NP, hd), lambda b, h: (b, 0, h)),
        out_shape=jax.ShapeDtypeStruct((BS, NP, D), F32),
    )(q, k, v, mask)


def _sa_call(qs, ks, vs):
    hd = 2 * DH
    return pl.pallas_call(
        _sa_body,
        grid=(BS, NH // 2),
        in_specs=[pl.BlockSpec((1, NP, hd), lambda b, h: (b, 0, h))] * 3,
        out_specs=pl.BlockSpec((1, NP, hd), lambda b, h: (b, 0, h)),
        out_shape=jax.ShapeDtypeStruct((BS, NP, D), F32),
    )(qs, ks, vs)


def kernel(tgt, memory, query_pos, key_pos, params):
    p = params

    x = _rows_call(_ln0_body,
                   [tgt, _w2(p["norm0"]["g"]), _w2(p["norm0"]["b"])])

    mrows = 512
    mn = pl.pallas_call(
        _mn_body,
        grid=(BS, MNP // mrows),
        in_specs=[pl.BlockSpec((1, mrows, D), lambda b, i: (b, i, 0))],
        out_specs=pl.BlockSpec((1, mrows, D), lambda b, i: (b, i, 0)),
        out_shape=jax.ShapeDtypeStruct((BS, MNP, D), F32),
    )(memory)

    for lp in p["layers"]:
        k, v = pl.pallas_call(
            _kv_body,
            grid=(BS, MNP // mrows),
            in_specs=[
                pl.BlockSpec((1, mrows, D), lambda b, i: (b, i, 0)),
                pl.BlockSpec((1, mrows, D), lambda b, i: (b, i, 0)),
                _full((D, D)), _full((1, D)), _full((D, D)), _full((1, D)),
            ],
            out_specs=[pl.BlockSpec((1, mrows, D), lambda b, i: (b, i, 0))] * 2,
            out_shape=[jax.ShapeDtypeStruct((BS, MNP, D), F32)] * 2,
        )(memory, key_pos, lp["l2"]["W"], _w2(lp["l2"]["b"]),
          lp["l3"]["W"], _w2(lp["l3"]["b"]))

        mcols = 1024
        sim = pl.pallas_call(
            _sim_body,
            grid=(BS, MNP // mcols),
            in_specs=[pl.BlockSpec((1, NP, D), lambda b, i: (b, 0, 0)),
                      pl.BlockSpec((1, mcols, D), lambda b, i: (b, i, 0))],
            out_specs=pl.BlockSpec((1, NP, mcols), lambda b, i: (b, 0, i)),
            out_shape=jax.ShapeDtypeStruct((BS, NP, MNP), F32),
        )(x, mn)
        mask = _rows_call(_topk_body, [sim], out_shape=(BS, NP, MNP))

        q = _rows_call(_q_body,
                       [x, query_pos, lp["l1"]["W"], _w2(lp["l1"]["b"])])
        att = _attn_call(q, k, v, mask)

        sa = lp["sa"]
        t1, qs, ks, vs = _rows_call(
            _post1_body,
            [x, att, query_pos, _w2(lp["norm1"]["g"]), _w2(lp["norm1"]["b"]),
             sa["Wq"], _w2(sa["bq"]), sa["Wk"], _w2(sa["bk"]),
             sa["Wv"], _w2(sa["bv"])],
            n_out=4)
        sao = _sa_call(qs, ks, vs)

        x = _rows_call(
            _post2_body,
            [t1, sao, sa["Wo"], _w2(sa["bo"]),
             _w2(lp["norm2"]["g"]), _w2(lp["norm2"]["b"]),
             lp["lin1"]["W"], _w2(lp["lin1"]["b"]),
             lp["lin2"]["W"], _w2(lp["lin2"]["b"]),
             _w2(lp["norm3"]["g"]), _w2(lp["norm3"]["b"])])

    ncp = 768
    cls_w = jnp.pad(p["cls_W"], ((0, 0), (0, ncp - NC)))
    bn = p["bn"]
    cls = _rows_call(
        _head_body,
        [x, _w2(bn["g"]), _w2(bn["b"]), _w2(bn["rm"]), _w2(bn["rv"]), cls_w],
        out_shape=(BS, NP, ncp))
    return (x, cls[..., :NC])


# speedup vs baseline: 7.9972x; 1.1073x over previous
import math

import jax
import jax.numpy as jnp
from jax import lax
from jax.experimental import pallas as pl

BS, NP, MNP, D, NH, DFF, NC, TOPK = 2, 256, 4096, 768, 12, 2048, 751, 32
DH = D // NH
F32 = jnp.float32
NEG = -1000000000.0


def _dot(a, b):
    return jnp.dot(a, b, preferred_element_type=F32)


def _dot_t(a, b):
    return lax.dot_general(a, b, (((1,), (1,)), ((), ())),
                           preferred_element_type=F32)


def _ln(x, g, b, eps=1e-5):
    mu = jnp.mean(x, axis=-1, keepdims=True)
    var = jnp.mean((x - mu) ** 2, axis=-1, keepdims=True)
    return (x - mu) / jnp.sqrt(var + eps) * g + b


def _w2(p):
    return p.reshape(1, -1)


def _ln0_body(x_ref, g_ref, b_ref, o_ref):
    o_ref[0] = _ln(x_ref[0], g_ref[...], b_ref[...])


def _mn_body(m_ref, mn_ref):
    m = m_ref[0]
    n = jnp.sqrt(jnp.sum(m * m, axis=1, keepdims=True))
    mn_ref[0] = m / n


def _kv_body(m_ref, kp_ref, w2_ref, b2_ref, w3_ref, b3_ref, k_ref, v_ref):
    m = m_ref[0]
    k_ref[0] = _dot(m + kp_ref[0], w2_ref[...]) + b2_ref[...]
    v_ref[0] = _dot(m, w3_ref[...]) + b3_ref[...]


def _seltopk_body(x_ref, mn_ref, qp_ref, w_ref, b_ref, mask_ref, q_ref):
    x = x_ref[0]
    xn = x / jnp.sqrt(jnp.sum(x * x, axis=1, keepdims=True))
    sim = _dot_t(xn, mn_ref[0])
    q_ref[0] = (_dot(x + qp_ref[0], w_ref[...]) + b_ref[...]) \
        * (1.0 / math.sqrt(DH))
    b = lax.bitcast_convert_type(sim, jnp.int32)
    key = b ^ (lax.shift_right_arithmetic(b, 31) & jnp.int32(0x7FFFFFFF))
    sign = jnp.int32(-2147483648)

    def body(_, carry):
        t_u, bit = carry
        cand = t_u | bit
        thr = cand ^ sign
        cnt = jnp.sum((key >= thr).astype(jnp.int32), axis=1, keepdims=True)
        return jnp.where(cnt >= TOPK, cand, t_u), lax.shift_right_logical(bit, 1)

    t_u, _ = lax.fori_loop(
        0, 32, body, (jnp.zeros((NP, 1), jnp.int32), jnp.full((), sign)))
    thr = t_u ^ sign
    gt = key > thr
    n_gt = jnp.sum(gt.astype(jnp.int32), axis=1, keepdims=True)
    eq = key == thr
    nb = MNP // 128
    eq2 = eq.astype(F32).reshape(NP * nb, 128)
    li = lax.broadcasted_iota(jnp.int32, (128, 128), 0)
    lj = lax.broadcasted_iota(jnp.int32, (128, 128), 1)
    inb = _dot(eq2, (li <= lj).astype(F32)).reshape(NP, nb, 128)
    btot = jnp.sum(eq.astype(F32).reshape(NP, nb, 128), axis=2)
    ci = lax.broadcasted_iota(jnp.int32, (nb, nb), 0)
    cj = lax.broadcasted_iota(jnp.int32, (nb, nb), 1)
    offs = _dot(btot, (ci < cj).astype(F32))
    incl = (inb + offs[:, :, None]).reshape(NP, MNP)
    need = (TOPK - n_gt).astype(F32)
    sel = eq & (incl <= need)
    mask_ref[0] = jnp.where(gt | sel, 0.0, NEG).astype(F32)


def _softmax(s):
    m = jnp.max(s, axis=1, keepdims=True)
    e = jnp.exp(s - m)
    return e / jnp.sum(e, axis=1, keepdims=True)


def _xattn_body(q_ref, k_ref, v_ref, mask_ref, o_ref):
    bias = mask_ref[0]
    outs = []
    for i in range(2):
        sl = slice(i * DH, (i + 1) * DH)
        s = _dot_t(q_ref[0][:, sl], k_ref[0][:, sl]) + bias
        outs.append(_dot(_softmax(s), v_ref[0][:, sl]))
    o_ref[0] = jnp.concatenate(outs, axis=1)


def _tail_body(x_ref, att_ref, qp_ref, g1_ref, b1_ref, wq_ref, bq_ref,
               wk_ref, bk_ref, wv_ref, bv_ref, wo_ref, bo_ref,
               g2_ref, b2_ref, w1_ref, bf1_ref, w2_ref, bf2_ref,
               g3_ref, b3_ref, o_ref):
    t1 = _ln(x_ref[0] + att_ref[0], g1_ref[...], b1_ref[...])
    qk = t1 + qp_ref[0]
    qs = (_dot(qk, wq_ref[...]) + bq_ref[...]) * (1.0 / math.sqrt(DH))
    ks = _dot(qk, wk_ref[...]) + bk_ref[...]
    vs = _dot(t1, wv_ref[...]) + bv_ref[...]
    outs = []
    for h in range(NH):
        sl = slice(h * DH, (h + 1) * DH)
        s = _dot_t(qs[:, sl], ks[:, sl])
        outs.append(_dot(_softmax(s), vs[:, sl]))
    sao = _dot(jnp.concatenate(outs, axis=1), wo_ref[...]) + bo_ref[...]
    t2 = _ln(t1 + sao, g2_ref[...], b2_ref[...])
    hh = _dot(t2, w1_ref[...]) + bf1_ref[...]
    hh = 0.5 * hh * (1.0 + lax.erf(hh * (1.0 / math.sqrt(2.0))))
    ff = _dot(hh, w2_ref[...]) + bf2_ref[...]
    o_ref[0] = _ln(t2 + ff, g3_ref[...], b3_ref[...])


def _head_body(x_ref, g_ref, b_ref, rm_ref, rv_ref, w_ref, c_ref):
    x = (x_ref[0] - rm_ref[...]) / jnp.sqrt(rv_ref[...] + 1e-5) \
        * g_ref[...] + b_ref[...]
    c_ref[0] = _dot(x, w_ref[...])


def _full(shape):
    nd = len(shape)
    return pl.BlockSpec(shape, lambda *_: (0,) * nd)


def _bmap(shape):
    nd = len(shape)
    return pl.BlockSpec(shape, lambda b, *_: (b,) + (0,) * (nd - 1))


def _rows_call(body, ins, n_out=1, out_shape=None):
    in_specs = [_bmap((1,) + a.shape[1:]) if a.shape[0] == BS else _full(a.shape)
                for a in ins]
    if out_shape is None:
        out_shape = (BS, NP, D)
    shapes = [out_shape] * n_out if isinstance(out_shape, tuple) else out_shape
    out_specs = [_bmap((1,) + s[1:]) for s in shapes]
    res = pl.pallas_call(
        body,
        grid=(BS,),
        in_specs=in_specs,
        out_specs=out_specs if n_out > 1 else out_specs[0],
        out_shape=[jax.ShapeDtypeStruct(s, F32) for s in shapes]
        if n_out > 1 else jax.ShapeDtypeStruct(shapes[0], F32),
    )(*ins)
    return res


def _attn_call(q, k, v, mask):
    hd = 2 * DH
    return pl.pallas_call(
        _xattn_body,
        grid=(BS, NH // 2),
        in_specs=[
            pl.BlockSpec((1, NP, hd), lambda b, h: (b, 0, h)),
            pl.BlockSpec((1, MNP, hd), lambda b, h: (b, 0, h)),
            pl.BlockSpec((1, MNP, hd), lambda b, h: (b, 0, h)),
            pl.BlockSpec((1, NP, MNP), lambda b, h: (b, 0, 0)),
        ],
        out_specs=pl.BlockSpec((1, NP, hd), lambda b, h: (b, 0, h)),
        out_shape=jax.ShapeDtypeStruct((BS, NP, D), F32),
    )(q, k, v, mask)


def kernel(tgt, memory, query_pos, key_pos, params):
    p = params

    x = _rows_call(_ln0_body,
                   [tgt, _w2(p["norm0"]["g"]), _w2(p["norm0"]["b"])])

    mrows = 512
    mn = pl.pallas_call(
        _mn_body,
        grid=(BS, MNP // mrows),
        in_specs=[pl.BlockSpec((1, mrows, D), lambda b, i: (b, i, 0))],
        out_specs=pl.BlockSpec((1, mrows, D), lambda b, i: (b, i, 0)),
        out_shape=jax.ShapeDtypeStruct((BS, MNP, D), F32),
    )(memory)

    for lp in p["layers"]:
        k, v = pl.pallas_call(
            _kv_body,
            grid=(BS, MNP // mrows),
            in_specs=[
                pl.BlockSpec((1, mrows, D), lambda b, i: (b, i, 0)),
                pl.BlockSpec((1, mrows, D), lambda b, i: (b, i, 0)),
                _full((D, D)), _full((1, D)), _full((D, D)), _full((1, D)),
            ],
            out_specs=[pl.BlockSpec((1, mrows, D), lambda b, i: (b, i, 0))] * 2,
            out_shape=[jax.ShapeDtypeStruct((BS, MNP, D), F32)] * 2,
        )(memory, key_pos, lp["l2"]["W"], _w2(lp["l2"]["b"]),
          lp["l3"]["W"], _w2(lp["l3"]["b"]))

        mask, q = pl.pallas_call(
            _seltopk_body,
            grid=(BS,),
            in_specs=[pl.BlockSpec((1, NP, D), lambda b: (b, 0, 0)),
                      pl.BlockSpec((1, MNP, D), lambda b: (b, 0, 0)),
                      pl.BlockSpec((1, NP, D), lambda b: (b, 0, 0)),
                      _full((D, D)), _full((1, D))],
            out_specs=[pl.BlockSpec((1, NP, MNP), lambda b: (b, 0, 0)),
                       pl.BlockSpec((1, NP, D), lambda b: (b, 0, 0))],
            out_shape=[jax.ShapeDtypeStruct((BS, NP, MNP), F32),
                       jax.ShapeDtypeStruct((BS, NP, D), F32)],
        )(x, mn, query_pos, lp["l1"]["W"], _w2(lp["l1"]["b"]))

        att = _attn_call(q, k, v, mask)

        sa = lp["sa"]
        x = _rows_call(
            _tail_body,
            [x, att, query_pos, _w2(lp["norm1"]["g"]), _w2(lp["norm1"]["b"]),
             sa["Wq"], _w2(sa["bq"]), sa["Wk"], _w2(sa["bk"]),
             sa["Wv"], _w2(sa["bv"]), sa["Wo"], _w2(sa["bo"]),
             _w2(lp["norm2"]["g"]), _w2(lp["norm2"]["b"]),
             lp["lin1"]["W"], _w2(lp["lin1"]["b"]),
             lp["lin2"]["W"], _w2(lp["lin2"]["b"]),
             _w2(lp["norm3"]["g"]), _w2(lp["norm3"]["b"])])

    ncp = 768
    cls_w = jnp.pad(p["cls_W"], ((0, 0), (0, ncp - NC)))
    bn = p["bn"]
    cls = _rows_call(
        _head_body,
        [x, _w2(bn["g"]), _w2(bn["b"]), _w2(bn["rm"]), _w2(bn["rv"]), cls_w],
        out_shape=(BS, NP, ncp))
    return (x, cls[..., :NC])


# speedup vs baseline: 8.1647x; 1.0209x over previous
import math

import jax
import jax.numpy as jnp
from jax import lax
from jax.experimental import pallas as pl

BS, NP, MNP, D, NH, DFF, NC, TOPK = 2, 256, 4096, 768, 12, 2048, 751, 32
DH = D // NH
F32 = jnp.float32
NEG = -1000000000.0


def _dot(a, b, prec=None):
    return jnp.dot(a, b, preferred_element_type=F32, precision=prec)


def _dot_t(a, b, prec=None):
    return lax.dot_general(a, b, (((1,), (1,)), ((), ())),
                           preferred_element_type=F32, precision=prec)


def _ln(x, g, b, eps=1e-5):
    mu = jnp.mean(x, axis=-1, keepdims=True)
    var = jnp.mean((x - mu) ** 2, axis=-1, keepdims=True)
    return (x - mu) / jnp.sqrt(var + eps) * g + b


def _w2(p):
    return p.reshape(1, -1)


def _ln0_body(x_ref, g_ref, b_ref, o_ref):
    o_ref[0] = _ln(x_ref[0], g_ref[...], b_ref[...])


def _kv_body(m_ref, kp_ref, w2_ref, b2_ref, w3_ref, b3_ref, k_ref, v_ref):
    m = m_ref[0]
    k_ref[0] = _dot(m + kp_ref[0], w2_ref[...]) + b2_ref[...]
    v_ref[0] = _dot(m, w3_ref[...]) + b3_ref[...]


def _seltopk_body(xn_ref, x_ref, mn_ref, qp_ref, w_ref, b_ref,
                  mask_ref, q_ref):
    x = x_ref[0]
    sim = _dot_t(xn_ref[0], mn_ref[0])
    q_ref[0] = (_dot(x + qp_ref[0], w_ref[...]) + b_ref[...]) \
        * (1.0 / math.sqrt(DH))
    b = lax.bitcast_convert_type(sim, jnp.int32)
    key = b ^ (lax.shift_right_arithmetic(b, 31) & jnp.int32(0x7FFFFFFF))
    sign = jnp.int32(-2147483648)

    def body(_, carry):
        t_u, bit = carry
        cand = t_u | bit
        thr = cand ^ sign
        cnt = jnp.sum((key >= thr).astype(jnp.int32), axis=1, keepdims=True)
        return jnp.where(cnt >= TOPK, cand, t_u), lax.shift_right_logical(bit, 1)

    t_u, _ = lax.fori_loop(
        0, 32, body, (jnp.zeros((NP, 1), jnp.int32), jnp.full((), sign)))
    thr = t_u ^ sign
    gt = key > thr
    n_gt = jnp.sum(gt.astype(jnp.int32), axis=1, keepdims=True)
    eq = key == thr
    nb = MNP // 128
    eq2 = eq.astype(F32).reshape(NP * nb, 128)
    li = lax.broadcasted_iota(jnp.int32, (128, 128), 0)
    lj = lax.broadcasted_iota(jnp.int32, (128, 128), 1)
    inb = _dot(eq2, (li <= lj).astype(F32)).reshape(NP, nb, 128)
    btot = jnp.sum(eq.astype(F32).reshape(NP, nb, 128), axis=2)
    ci = lax.broadcasted_iota(jnp.int32, (nb, nb), 0)
    cj = lax.broadcasted_iota(jnp.int32, (nb, nb), 1)
    offs = _dot(btot, (ci < cj).astype(F32))
    incl = (inb + offs[:, :, None]).reshape(NP, MNP)
    need = (TOPK - n_gt).astype(F32)
    sel = eq & (incl <= need)
    mask_ref[0] = jnp.where(gt | sel, 0.0, NEG).astype(F32)


def _sm_dot(s, v):
    m = jnp.max(s, axis=1, keepdims=True)
    e = jnp.exp(s - m)
    denom = jnp.sum(e, axis=1, keepdims=True)
    return _dot(e, v) / denom


def _xattn_body(q_ref, k_ref, v_ref, mask_ref, o_ref):
    bias = mask_ref[0]
    outs = []
    for i in range(2):
        sl = slice(i * DH, (i + 1) * DH)
        s = _dot_t(q_ref[0][:, sl], k_ref[0][:, sl]) + bias
        outs.append(_sm_dot(s, v_ref[0][:, sl]))
    o_ref[0] = jnp.concatenate(outs, axis=1)


def _tail_body(x_ref, att_ref, qp_ref, g1_ref, b1_ref, wq_ref, bq_ref,
               wk_ref, bk_ref, wv_ref, bv_ref, wo_ref, bo_ref,
               g2_ref, b2_ref, w1_ref, bf1_ref, w2_ref, bf2_ref,
               g3_ref, b3_ref, o_ref):
    t1 = _ln(x_ref[0] + att_ref[0], g1_ref[...], b1_ref[...])
    qk = t1 + qp_ref[0]
    qs = (_dot(qk, wq_ref[...]) + bq_ref[...]) * (1.0 / math.sqrt(DH))
    ks = _dot(qk, wk_ref[...]) + bk_ref[...]
    vs = _dot(t1, wv_ref[...]) + bv_ref[...]
    outs = []
    for h in range(NH):
        sl = slice(h * DH, (h + 1) * DH)
        s = _dot_t(qs[:, sl], ks[:, sl])
        outs.append(_sm_dot(s, vs[:, sl]))
    sao = _dot(jnp.concatenate(outs, axis=1), wo_ref[...]) + bo_ref[...]
    t2 = _ln(t1 + sao, g2_ref[...], b2_ref[...])
    hh = _dot(t2, w1_ref[...]) + bf1_ref[...]
    hh = 0.5 * hh * (1.0 + lax.erf(hh * (1.0 / math.sqrt(2.0))))
    ff = _dot(hh, w2_ref[...]) + bf2_ref[...]
    o_ref[0] = _ln(t2 + ff, g3_ref[...], b3_ref[...])


def _head_body(x_ref, g_ref, b_ref, rm_ref, rv_ref, w_ref, c_ref):
    x = (x_ref[0] - rm_ref[...]) / jnp.sqrt(rv_ref[...] + 1e-5) \
        * g_ref[...] + b_ref[...]
    c_ref[0] = _dot(x, w_ref[...])


def _full(shape):
    nd = len(shape)
    return pl.BlockSpec(shape, lambda *_: (0,) * nd)


def _bmap(shape):
    nd = len(shape)
    return pl.BlockSpec(shape, lambda b, *_: (b,) + (0,) * (nd - 1))


def _rows_call(body, ins, n_out=1, out_shape=None):
    in_specs = [_bmap((1,) + a.shape[1:]) if a.shape[0] == BS else _full(a.shape)
                for a in ins]
    if out_shape is None:
        out_shape = (BS, NP, D)
    shapes = [out_shape] * n_out if isinstance(out_shape, tuple) else out_shape
    out_specs = [_bmap((1,) + s[1:]) for s in shapes]
    res = pl.pallas_call(
        body,
        grid=(BS,),
        in_specs=in_specs,
        out_specs=out_specs if n_out > 1 else out_specs[0],
        out_shape=[jax.ShapeDtypeStruct(s, F32) for s in shapes]
        if n_out > 1 else jax.ShapeDtypeStruct(shapes[0], F32),
    )(*ins)
    return res


def _attn_call(q, k, v, mask):
    hd = 2 * DH
    return pl.pallas_call(
        _xattn_body,
        grid=(BS, NH // 2),
        in_specs=[
            pl.BlockSpec((1, NP, hd), lambda b, h: (b, 0, h)),
            pl.BlockSpec((1, MNP, hd), lambda b, h: (b, 0, h)),
            pl.BlockSpec((1, MNP, hd), lambda b, h: (b, 0, h)),
            pl.BlockSpec((1, NP, MNP), lambda b, h: (b, 0, 0)),
        ],
        out_specs=pl.BlockSpec((1, NP, hd), lambda b, h: (b, 0, h)),
        out_shape=jax.ShapeDtypeStruct((BS, NP, D), F32),
    )(q, k, v, mask)


def kernel(tgt, memory, query_pos, key_pos, params):
    p = params

    x = _rows_call(_ln0_body,
                   [tgt, _w2(p["norm0"]["g"]), _w2(p["norm0"]["b"])])

    mn = memory / jnp.linalg.norm(memory, ord=2, axis=2, keepdims=True)
    mrows = 512

    for lp in p["layers"]:
        xn = x / jnp.linalg.norm(x, ord=2, axis=2, keepdims=True)
        k, v = pl.pallas_call(
            _kv_body,
            grid=(BS, MNP // mrows),
            in_specs=[
                pl.BlockSpec((1, mrows, D), lambda b, i: (b, i, 0)),
                pl.BlockSpec((1, mrows, D), lambda b, i: (b, i, 0)),
                _full((D, D)), _full((1, D)), _full((D, D)), _full((1, D)),
            ],
            out_specs=[pl.BlockSpec((1, mrows, D), lambda b, i: (b, i, 0))] * 2,
            out_shape=[jax.ShapeDtypeStruct((BS, MNP, D), F32)] * 2,
        )(memory, key_pos, lp["l2"]["W"], _w2(lp["l2"]["b"]),
          lp["l3"]["W"], _w2(lp["l3"]["b"]))

        mask, q = pl.pallas_call(
            _seltopk_body,
            grid=(BS,),
            in_specs=[pl.BlockSpec((1, NP, D), lambda b: (b, 0, 0)),
                      pl.BlockSpec((1, NP, D), lambda b: (b, 0, 0)),
                      pl.BlockSpec((1, MNP, D), lambda b: (b, 0, 0)),
                      pl.BlockSpec((1, NP, D), lambda b: (b, 0, 0)),
                      _full((D, D)), _full((1, D))],
            out_specs=[pl.BlockSpec((1, NP, MNP), lambda b: (b, 0, 0)),
                       pl.BlockSpec((1, NP, D), lambda b: (b, 0, 0))],
            out_shape=[jax.ShapeDtypeStruct((BS, NP, MNP), F32),
                       jax.ShapeDtypeStruct((BS, NP, D), F32)],
        )(xn, x, mn, query_pos, lp["l1"]["W"], _w2(lp["l1"]["b"]))

        att = _attn_call(q, k, v, mask)

        sa = lp["sa"]
        x = _rows_call(
            _tail_body,
            [x, att, query_pos, _w2(lp["norm1"]["g"]), _w2(lp["norm1"]["b"]),
             sa["Wq"], _w2(sa["bq"]), sa["Wk"], _w2(sa["bk"]),
             sa["Wv"], _w2(sa["bv"]), sa["Wo"], _w2(sa["bo"]),
             _w2(lp["norm2"]["g"]), _w2(lp["norm2"]["b"]),
             lp["lin1"]["W"], _w2(lp["lin1"]["b"]),
             lp["lin2"]["W"], _w2(lp["lin2"]["b"]),
             _w2(lp["norm3"]["g"]), _w2(lp["norm3"]["b"])])

    ncp = 768
    cls_w = jnp.pad(p["cls_W"], ((0, 0), (0, ncp - NC)))
    bn = p["bn"]
    cls = _rows_call(
        _head_body,
        [x, _w2(bn["g"]), _w2(bn["b"]), _w2(bn["rm"]), _w2(bn["rv"]), cls_w],
        out_shape=(BS, NP, ncp))
    return (x, cls[..., :NC])
